# baseline (speedup 1.0000x reference)
import functools

import jax
import jax.numpy as jnp
from jax import lax
from jax.experimental import pallas as pl
from jax.experimental.pallas import tpu as pltpu
from jax.experimental.pallas import tpu_sc as plsc

N_CLASSES = 1000
FEAT = 128
BATCH = 16384
DECAY = 0.3

NC, NS, L = 2, 16, 16
NW = NC * NS
CP = 1024
RPT = BATCH // NW
CHUNK = 128
NCHUNK = RPT // CHUNK
STRIPE = CP // NS
CW = 16
ZROWS = 8

_mesh = plsc.VectorSubcoreMesh(
    core_axis_name="c", subcore_axis_name="s", num_cores=NC, num_subcores=NS)


@functools.partial(
    pl.kernel,
    out_type=(
        jax.ShapeDtypeStruct((NC, CP, FEAT), jnp.float32),
        jax.ShapeDtypeStruct((NC, CP, CW), jnp.float32),
    ),
    mesh=_mesh,
    compiler_params=pltpu.CompilerParams(needs_layout_passes=False),
    scratch_types=[
        pltpu.VMEM((NCHUNK, CHUNK), jnp.int32),
        pltpu.VMEM((RPT, FEAT), jnp.float32),
        pltpu.VMEM((CP,), jnp.float32),
        pltpu.VMEM((ZROWS, FEAT), jnp.float32),
        pltpu.VMEM((NS, STRIPE), jnp.float32),
        pltpu.VMEM((STRIPE, CW), jnp.float32),
        pltpu.VMEM_SHARED((CP, FEAT), jnp.float32),
        pltpu.VMEM_SHARED((NS, CP), jnp.float32),
        pltpu.SemaphoreType.DMA,
        pltpu.SemaphoreType.DMA,
    ],
)
def _accumulate(x_hbm, y_hbm, sums_hbm, cnts_hbm,
                idx_v, xb_v, h_v, zs_v, red_v, cb_v, ssum, stage, sem, sem2):
    cid = lax.axis_index("c")
    sid = lax.axis_index("s")
    wid = cid * NS + sid

    zero = jnp.zeros((L,), jnp.float32)
    one = jnp.ones((L,), jnp.float32)

    for i in range(ZROWS):
        for j in range(FEAT // L):
            zs_v[i, pl.ds(j * L, L)] = zero
    for j in range(CP // L):
        h_v[pl.ds(j * L, L)] = zero

    zds = [pltpu.async_copy(
        zs_v, ssum.at[pl.ds(sid * STRIPE + k * ZROWS, ZROWS)], sem)
        for k in range(STRIPE // ZROWS)]
    pltpu.sync_copy(y_hbm.at[pl.ds(wid * NCHUNK, NCHUNK)], idx_v)
    for d in zds:
        d.wait()
    plsc.subcore_barrier()

    loads = [pltpu.async_copy(
        x_hbm.at[pl.ds(wid * RPT + j * CHUNK, CHUNK)],
        xb_v.at[pl.ds(j * CHUNK, CHUNK)], sem) for j in range(NCHUNK)]
    for j in range(NCHUNK):
        for k in range(CHUNK // L):
            iv = idx_v[j, pl.ds(k * L, L)]
            plsc.addupdate_scatter(h_v, [iv], one)

    scs = []
    for j in range(NCHUNK):
        loads[j].wait()
        scs.append(pltpu.async_copy(
            xb_v.at[pl.ds(j * CHUNK, CHUNK)],
            ssum.at[idx_v.at[j]], sem2, add=True))
    for d in scs:
        d.wait()

    pltpu.sync_copy(h_v, stage.at[sid])
    plsc.subcore_barrier()

    rows = pl.ds(sid * STRIPE, STRIPE)
    sums_done = pltpu.async_copy(ssum.at[rows], sums_hbm.at[cid, rows], sem)

    rds = [pltpu.async_copy(stage.at[i, pl.ds(sid * STRIPE, STRIPE)],
                            red_v.at[i], sem)
           for i in range(NS)]
    for d in rds:
        d.wait()
    lanes = lax.broadcasted_iota(jnp.int32, (L,), 0)
    for g in range(STRIPE // L):
        acc = red_v[0, pl.ds(g * L, L)]
        for i in range(1, NS):
            acc = acc + red_v[i, pl.ds(g * L, L)]
        rows_idx = lanes + g * L
        for j in range(CW):
            plsc.store_scatter(
                cb_v, [rows_idx, jnp.full((L,), j, jnp.int32)], acc)
    pltpu.sync_copy(cb_v, cnts_hbm.at[cid, rows])
    sums_done.wait()


def _tc_combine_body(s_ref, c_ref, cen_ref, o_ref):
    s = s_ref[0] + s_ref[1]
    cnt = c_ref[0, :, :1] + c_ref[1, :, :1]
    present = cnt > 0.0
    inv = 1.0 / jnp.where(present, cnt, 1.0)
    old = cen_ref[...]
    upd = s * inv * DECAY + (1.0 - DECAY) * old
    o_ref[...] = jnp.where(present, upd, old)


_TCB = 200

_combine = pl.pallas_call(
    _tc_combine_body,
    grid=(N_CLASSES // _TCB,),
    in_specs=[
        pl.BlockSpec((NC, _TCB, FEAT), lambda i: (0, i, 0)),
        pl.BlockSpec((NC, _TCB, CW), lambda i: (0, i, 0)),
        pl.BlockSpec((_TCB, FEAT), lambda i: (i, 0)),
    ],
    out_specs=pl.BlockSpec((_TCB, FEAT), lambda i: (i, 0)),
    out_shape=jax.ShapeDtypeStruct((N_CLASSES, FEAT), jnp.float32),
)


def kernel(x, centroids, y):
    y2 = y.astype(jnp.int32).reshape(NW * NCHUNK, CHUNK)
    sums, cnts = _accumulate(x, y2)
    return _combine(sums, cnts, centroids)


# speedup vs baseline: 1.1029x; 1.1029x over previous
import functools

import jax
import jax.numpy as jnp
from jax import lax
from jax.experimental import pallas as pl
from jax.experimental.pallas import tpu as pltpu
from jax.experimental.pallas import tpu_sc as plsc

N_CLASSES = 1000
FEAT = 128
BATCH = 16384
DECAY = 0.3

NC, NS, L = 2, 16, 16
NW = NC * NS
CP = 1024
RPT = BATCH // NW
CHUNK = 128
NCHUNK = RPT // CHUNK
STRIPE = CP // NS
ZROWS = 8

_mesh = plsc.VectorSubcoreMesh(
    core_axis_name="c", subcore_axis_name="s", num_cores=NC, num_subcores=NS)


@functools.partial(
    pl.kernel,
    out_type=(
        jax.ShapeDtypeStruct((NC, CP, FEAT), jnp.float32),
        jax.ShapeDtypeStruct((NC, NS, CP), jnp.float32),
    ),
    mesh=_mesh,
    compiler_params=pltpu.CompilerParams(needs_layout_passes=False),
    scratch_types=[
        pltpu.VMEM((NCHUNK, CHUNK), jnp.int32),
        pltpu.VMEM((RPT, FEAT), jnp.float32),
        pltpu.VMEM((CP,), jnp.float32),
        pltpu.VMEM((ZROWS, FEAT), jnp.float32),
        pltpu.VMEM_SHARED((CP, FEAT), jnp.float32),
        pltpu.SemaphoreType.DMA,
        pltpu.SemaphoreType.DMA,
    ],
)
def _accumulate(x_hbm, y_hbm, sums_hbm, hist_hbm,
                idx_v, xb_v, h_v, zs_v, ssum, sem, sem2):
    cid = lax.axis_index("c")
    sid = lax.axis_index("s")
    wid = cid * NS + sid

    zero = jnp.zeros((L,), jnp.float32)
    one = jnp.ones((L,), jnp.float32)

    for i in range(ZROWS):
        for j in range(FEAT // L):
            zs_v[i, pl.ds(j * L, L)] = zero
    for j in range(CP // L):
        h_v[pl.ds(j * L, L)] = zero

    zds = [pltpu.async_copy(
        zs_v, ssum.at[pl.ds(sid * STRIPE + k * ZROWS, ZROWS)], sem)
        for k in range(STRIPE // ZROWS)]
    pltpu.sync_copy(y_hbm.at[pl.ds(wid * NCHUNK, NCHUNK)], idx_v)
    for d in zds:
        d.wait()
    plsc.subcore_barrier()

    loads = [pltpu.async_copy(
        x_hbm.at[pl.ds(wid * RPT + j * CHUNK, CHUNK)],
        xb_v.at[pl.ds(j * CHUNK, CHUNK)], sem) for j in range(NCHUNK)]
    for j in range(NCHUNK):
        for k in range(CHUNK // L):
            iv = idx_v[j, pl.ds(k * L, L)]
            plsc.addupdate_scatter(h_v, [iv], one)

    scs = []
    for j in range(NCHUNK):
        loads[j].wait()
        scs.append(pltpu.async_copy(
            xb_v.at[pl.ds(j * CHUNK, CHUNK)],
            ssum.at[idx_v.at[j]], sem2, add=True))
    hist_done = pltpu.async_copy(h_v, hist_hbm.at[cid, sid], sem)
    for d in scs:
        d.wait()
    plsc.subcore_barrier()

    rows = pl.ds(sid * STRIPE, STRIPE)
    pltpu.sync_copy(ssum.at[rows], sums_hbm.at[cid, rows])
    hist_done.wait()


def _tc_combine_body(s_ref, h_ref, cen_ref, o_ref):
    s = s_ref[0, :N_CLASSES, :] + s_ref[1, :N_CLASSES, :]
    cnt = (jnp.sum(h_ref[0], axis=0)
           + jnp.sum(h_ref[1], axis=0))[:N_CLASSES, None]
    present = cnt > 0.0
    inv = 1.0 / jnp.where(present, cnt, 1.0)
    old = cen_ref[...]
    upd = s * inv * DECAY + (1.0 - DECAY) * old
    o_ref[...] = jnp.where(present, upd, old)


_combine = pl.pallas_call(
    _tc_combine_body,
    out_shape=jax.ShapeDtypeStruct((N_CLASSES, FEAT), jnp.float32),
)


def kernel(x, centroids, y):
    y2 = y.astype(jnp.int32).reshape(NW * NCHUNK, CHUNK)
    sums, hist = _accumulate(x, y2)
    return _combine(sums, hist, centroids)
